# baseline (speedup 1.0000x reference)
import functools

import jax
import jax.numpy as jnp
from jax import lax
from jax.experimental import pallas as pl
from jax.experimental.pallas import tpu as pltpu
from jax.experimental.pallas import tpu_sc as plsc

D_MODEL = 64
SCALE = 8.0
NC, NS = 2, 16
NW = NC * NS
BB = 128
LANES = 16
TILE = 8 * BB
H = 500096


def _pack_table(table):
    tt = table.T

    def body(a_ref, b_ref, o_ref):
        o_ref[:, :D_MODEL] = a_ref[...].T
        o_ref[:, D_MODEL:] = b_ref[...].T

    return pl.pallas_call(
        body,
        grid=(H // BB,),
        in_specs=[
            pl.BlockSpec((D_MODEL, BB), lambda i: (0, i)),
            pl.BlockSpec((D_MODEL, BB), lambda i: (0, H // BB + i)),
        ],
        out_specs=pl.BlockSpec((BB, 2 * D_MODEL), lambda i: (i, 0)),
        out_shape=jax.ShapeDtypeStruct((H, 2 * D_MODEL), jnp.float32),
    )(tt, tt)


def kernel(x, table):
    B0, S = x.shape
    V = table.shape[0]
    assert B0 == NW * BB and D_MODEL == table.shape[1] and V <= 2 * H
    xt = x.T
    if xt.dtype != jnp.int32:
        xt = xt.astype(jnp.int32)
    table2 = _pack_table(table)

    mesh = plsc.VectorSubcoreMesh(core_axis_name="c", subcore_axis_name="s")

    @functools.partial(
        pl.kernel,
        mesh=mesh,
        out_type=jax.ShapeDtypeStruct((S, 8, NW, TILE), jnp.float32),
        scratch_types=[
            pltpu.VMEM((S, BB), jnp.int32),
            pltpu.VMEM((BB, 2 * D_MODEL), jnp.float32),
            pltpu.VMEM((BB, 2 * D_MODEL), jnp.float32),
            pltpu.VMEM((D_MODEL * BB,), jnp.float32),
            pltpu.VMEM((D_MODEL * BB,), jnp.float32),
            pltpu.VMEM((BB,), jnp.int32),
            pltpu.VMEM((BB,), jnp.int32),
            pltpu.VMEM((BB,), jnp.int32),
            pltpu.SemaphoreType.DMA,
            pltpu.SemaphoreType.DMA,
            pltpu.SemaphoreType.DMA,
            pltpu.SemaphoreType.DMA,
        ],
        compiler_params=pltpu.CompilerParams(
            use_tc_tiling_on_sc=False, needs_layout_passes=False),
    )
    def emb(xt_hbm, table2_hbm, out_hbm, idx_v, rows0, rows1, trans0, trans1,
            idxs0, idxs1, par_v, gsem0, gsem1, osem0, osem1):
        rows_bufs = (rows0, rows1)
        trans_bufs = (trans0, trans1)
        idxs_bufs = (idxs0, idxs1)
        gsems = (gsem0, gsem1)
        osems = (osem0, osem1)
        wid = lax.axis_index("s") * NC + lax.axis_index("c")
        pltpu.sync_copy(xt_hbm.at[:, pl.ds(wid * BB, BB)], idx_v)

        iota = lax.iota(jnp.int32, LANES)
        dcol = [iota + c * LANES for c in range(D_MODEL // LANES)]
        ddst = [(iota + c * LANES) * BB for c in range(D_MODEL // LANES)]

        def prep_idxs(s, p):
            for k in range(BB // LANES):
                sl = pl.ds(k * LANES, LANES)
                v = idx_v[s, sl]
                idxs_bufs[p][sl] = jnp.where(v >= H, v - H, v)

        def fire(p):
            pltpu.async_copy(
                table2_hbm.at[idxs_bufs[p]], rows_bufs[p], gsems[p])

        def out_copies(s, p, wait):
            trans = trans_bufs[p]
            for td in range(8):
                cp = pltpu.make_async_copy(
                    trans.at[pl.ds(td * TILE, TILE)],
                    out_hbm.at[s, td, wid], osems[p])
                if wait:
                    cp.wait()
                else:
                    cp.start()

        for p in range(2):
            prep_idxs(p, p)
            fire(p)

        def step(i, carry):
            for p in range(2):
                s = i * 2 + p
                rows = rows_bufs[p]
                trans = trans_bufs[p]
                pltpu.make_async_copy(
                    table2_hbm.at[idxs_bufs[p]], rows, gsems[p]).wait()

                for k in range(BB // LANES):
                    sl = pl.ds(k * LANES, LANES)
                    par_v[sl] = jnp.where(idx_v[s, sl] >= H, 1, 0)

                @pl.when(s >= 2)
                def _(s=s, p=p):
                    out_copies(s - 2, p, wait=True)

                @plsc.parallel_loop(0, LANES, unroll=1)
                def _(t, rows=rows, trans=trans):
                    a = (iota + t) & 15
                    for q in range(BB // LANES):
                        bi = a + q * LANES
                        par = plsc.load_gather(par_v, [bi])
                        pcol = par * D_MODEL
                        for c in range(D_MODEL // LANES):
                            vals = plsc.load_gather(
                                rows, [bi, pcol + dcol[c]])
                            plsc.store_scatter(
                                trans, [ddst[c] + bi], vals * SCALE)

                out_copies(s, p, wait=False)

                @pl.when(s + 2 < S)
                def _(s=s, p=p):
                    prep_idxs(s + 2, p)
                    fire(p)

            return carry

        lax.fori_loop(0, S // 2, step, 0)
        for p in range(2):
            out_copies(S - 2 + p, p, wait=True)

    out5 = emb(xt, table2)
    out5 = out5.reshape(S, 8, NW, 8, BB)
    return out5.transpose(2, 4, 0, 1, 3).reshape(B0, S, D_MODEL)
